# baseline (speedup 1.0000x reference)
import functools
import jax
import jax.numpy as jnp
from jax import lax
from jax.experimental import pallas as pl
from jax.experimental.pallas import tpu as pltpu
from jax.experimental.pallas import tpu_sc as plsc

_NU = 50000
_NI = 50000
_N = 100000
_NP = 102400
_D = 64
_NG = 4
_GC = 16
_NLAYERS = 3
_B = 4096
_NNZ = 800000
_CK = 128
_NCHUNK = _NNZ // _CK
_NS = 16
_PT = _NP // _NS
_EC = 160
_ZR = 1280
_TJ = 400
_NSP = 100352
_PTS = _NSP // _NS

_MESH = plsc.VectorSubcoreMesh(core_axis_name="c", subcore_axis_name="s")
_MAGIC = 0x5F3759DF


def _fill_zero_rows(buf, nrows):
    def body(r, _):
        buf[r, :] = jnp.zeros((16,), jnp.float32)
        return 0
    lax.fori_loop(0, nrows, body, 0)


def _rsqrt16(x):
    bits = plsc.bitcast(x, jnp.int32)
    y = plsc.bitcast(_MAGIC - lax.shift_right_logical(bits, 1), jnp.float32)
    for _ in range(3):
        y = y * (1.5 - 0.5 * x * y * y)
    return y


@functools.partial(
    pl.kernel,
    mesh=_MESH,
    compiler_params=pltpu.CompilerParams(use_tc_tiling_on_sc=False, needs_layout_passes=False),
    out_type=(
        jax.ShapeDtypeStruct((_NP,), jnp.float32),
        jax.ShapeDtypeStruct((_NP,), jnp.float32),
        jax.ShapeDtypeStruct((_NG * _N, _GC), jnp.float32),
    ),
    scratch_types=(
        [pltpu.VMEM_SHARED((_NP,), jnp.float32),
         pltpu.VMEM_SHARED((_NP,), jnp.float32)]
        + [pltpu.VMEM((2, _CK), jnp.int32) for _ in range(4)]
        + [pltpu.VMEM((_CK,), jnp.int32) for _ in range(4)]
        + [pltpu.SemaphoreType.DMA for _ in range(4)]
        + [
            pltpu.VMEM((_CK,), jnp.float32),
            pltpu.VMEM((_ZR,), jnp.float32),
            pltpu.VMEM((_PT,), jnp.float32),
            pltpu.VMEM((_PT,), jnp.float32),
            pltpu.VMEM((_PT,), jnp.float32),
            pltpu.VMEM((1008, _GC), jnp.float32),
            pltpu.VMEM((1008, _GC), jnp.float32),
            pltpu.VMEM((1008,), jnp.float32),
        ]
    ),
)
def _prep_kernel(adj_t, e0g, dinv_o, dinv2_o, t0_o,
                 deg_sp, dinv_sp, ib0, ib1, ib2, ib3,
                 aN0, aN1, aN2, aN3, sS0, sS1, sS2, sS3, ones_v, zb,
                 degb, dvb, d2b, eb, tb, dv1k):
    c = lax.axis_index("c")
    s = lax.axis_index("s")

    @pl.when(c == 0)
    def _():
        def zrow(r, _):
            zb[pl.ds(r * 16, 16)] = jnp.zeros((16,), jnp.float32)
            return 0
        lax.fori_loop(0, _ZR // 16, zrow, 0)
        for k in range(_PT // _ZR):
            pltpu.sync_copy(zb, deg_sp.at[pl.ds(s * _PT + k * _ZR, _ZR)])
        for k in range(_CK // 16):
            ones_v[pl.ds(k * 16, 16)] = jnp.ones((16,), jnp.float32)
        plsc.subcore_barrier()

        ib = (ib0, ib1, ib2, ib3)
        aN = (aN0, aN1, aN2, aN3)
        sS = (sS0, sS1, sS2, sS3)

        def load_x(j, sl):
            pltpu.sync_copy(adj_t.at[s * _TJ + j], ib[sl])
            for k in range(_CK // 16):
                ssl = pl.ds(k * 16, 16)
                aN[sl][ssl] = ib[sl][1, ssl] + _NU

        def fire_s(sl):
            pltpu.async_copy(ones_v, deg_sp.at[ib[sl].at[0]], sS[sl], add=True)
            pltpu.async_copy(ones_v, deg_sp.at[aN[sl]], sS[sl], add=True)

        def wait_s(sl):
            pltpu.make_async_copy(ones_v, deg_sp.at[ib[sl].at[0]], sS[sl]).wait()
            pltpu.make_async_copy(ones_v, deg_sp.at[aN[sl]], sS[sl]).wait()

        for u in range(4):
            load_x(u, u)
            fire_s(u)

        def main_body(kk, _):
            for u in range(4):
                wait_s(u)
                load_x(4 + 4 * kk + u, u)
                fire_s(u)
            return 0
        lax.fori_loop(0, (_TJ - 4) // 4, main_body, 0)
        for u in range(4):
            wait_s(u)
        plsc.subcore_barrier()

        base = s * _PT
        pltpu.sync_copy(deg_sp.at[pl.ds(base, _PT)], degb)

        def unit(k, _):
            sl = pl.ds(k * 16, 16)
            v = degb[sl]
            pos = v > 0.0
            x = jnp.maximum(v, 1.0)
            y = _rsqrt16(x)
            dvb[sl] = jnp.where(pos, y, 0.0)
            d2b[sl] = jnp.where(pos, 1.0 / x, 0.0)
            return 0
        lax.fori_loop(0, _PT // 16, unit, 0)
        pltpu.sync_copy(dvb, dinv_o.at[pl.ds(base, _PT)])
        pltpu.sync_copy(d2b, dinv2_o.at[pl.ds(base, _PT)])
        pltpu.sync_copy(dvb, dinv_sp.at[pl.ds(base, _PT)])
        plsc.subcore_barrier()

        def t0_chunk(j, _):
            rb = s * 25000 + j * 1000
            n0 = lax.rem(rb, _N)
            pltpu.sync_copy(e0g.at[pl.ds(rb, 1000)], eb.at[pl.ds(0, 1000)])
            pltpu.sync_copy(dinv_sp.at[pl.ds(n0, 1000)], dv1k.at[pl.ds(0, 1000)])

            def grp(q, _):
                dvv = dv1k[pl.ds(q * 16, 16)]
                for tt in range(16):
                    r = q * 16 + tt
                    tb[r, :] = eb[r, :] * dvv[tt]
                return 0
            lax.fori_loop(0, 1000 // 16 + 1, grp, 0)
            pltpu.sync_copy(tb.at[pl.ds(0, 1000)], t0_o.at[pl.ds(rb, 1000)])
            return 0
        lax.fori_loop(0, 25, t0_chunk, 0)


@functools.partial(
    pl.kernel,
    mesh=_MESH,
    compiler_params=pltpu.CompilerParams(use_tc_tiling_on_sc=False, needs_layout_passes=False),
    out_type=(
        jax.ShapeDtypeStruct((_NG * _N, _GC), jnp.float32),
        jax.ShapeDtypeStruct((_NG * _N, _GC), jnp.float32),
    ),
    scratch_types=(
        [pltpu.VMEM_SHARED((_NSP, _GC), jnp.float32)]
        + [pltpu.VMEM((2, 2, _CK), jnp.int32) for _ in range(3)]
        + [pltpu.VMEM((_CK,), jnp.int32) for _ in range(16)]
        + [pltpu.VMEM((_CK, _GC), jnp.float32) for _ in range(8)]
        + [pltpu.SemaphoreType.DMA for _ in range(11)]
        + [
            pltpu.VMEM((_EC, _GC), jnp.float32),
            pltpu.VMEM((_EC, _GC), jnp.float32),
            pltpu.VMEM((_EC, _GC), jnp.float32),
            pltpu.VMEM((_EC,), jnp.float32),
            pltpu.VMEM((_EC,), jnp.float32),
        ]
    ),
)
def _layer_kernel(adj_t, t4, acc4, dinv_h, dinv2_h,
                  tn_o, accn_o, msg_sp,
                  ip0, ip1, ip2,
                  aB0, aB1, aB2, aB3, aN0, aN1, aN2, aN3,
                  gA0, gA1, gA2, gA3, gB0, gB1, gB2, gB3,
                  rA0, rA1, rA2, rA3, rB0, rB1, rB2, rB3,
                  sI0, sI1, sI2, sG0, sG1, sG2, sG3, sS0, sS1, sS2, sS3,
                  mb, ab, tnb, dv, d2):
    c = lax.axis_index("c")
    s = lax.axis_index("s")
    ip = (ip0, ip1, ip2)
    aB = (aB0, aB1, aB2, aB3)
    aN = (aN0, aN1, aN2, aN3)
    gA = (gA0, gA1, gA2, gA3)
    gB = (gB0, gB1, gB2, gB3)
    rA = (rA0, rA1, rA2, rA3)
    rB = (rB0, rB1, rB2, rB3)
    sI = (sI0, sI1, sI2)
    sG = (sG0, sG1, sG2, sG3)
    sS = (sS0, sS1, sS2, sS3)
    clamp = _NG * _N - 1

    for p in range(2):
        g = c * 2 + p
        gbase = g * _N

        _fill_zero_rows(rA0, _CK)

        def zcopy(k, _):
            pltpu.sync_copy(rA0, msg_sp.at[pl.ds(s * _PTS + k * _CK, _CK)])
            return 0
        lax.fori_loop(0, _PTS // _CK, zcopy, 0)
        plsc.subcore_barrier()

        def xf(p3, q, sl):
            for k in range(_CK // 16):
                ssl = pl.ds(k * 16, 16)
                av = ip[p3][q, 0, ssl]
                iv = ip[p3][q, 1, ssl]
                aB[sl][ssl] = av
                gA[sl][ssl] = jnp.minimum(av + gbase, clamp)
                aN[sl][ssl] = iv + _NU
                gB[sl][ssl] = jnp.minimum(iv + (gbase + _NU), clamp)
            pltpu.async_copy(t4.at[gA[sl]], rA[sl], sG[sl])
            pltpu.async_copy(t4.at[gB[sl]], rB[sl], sG[sl])

        def wait_g(sl):
            pltpu.make_async_copy(t4.at[gA[sl]], rA[sl], sG[sl]).wait()
            pltpu.make_async_copy(t4.at[gB[sl]], rB[sl], sG[sl]).wait()

        def fire_s(sl):
            pltpu.async_copy(rA[sl], msg_sp.at[aN[sl]], sS[sl], add=True)
            pltpu.async_copy(rB[sl], msg_sp.at[aB[sl]], sS[sl], add=True)

        def wait_s(sl):
            pltpu.make_async_copy(rA[sl], msg_sp.at[aN[sl]], sS[sl]).wait()
            pltpu.make_async_copy(rB[sl], msg_sp.at[aB[sl]], sS[sl]).wait()

        def fire_pair(b2, p3):
            pltpu.async_copy(adj_t.at[pl.ds(s * _TJ + b2 * 2, 2)], ip[p3], sI[p3])

        def wait_pair(p3):
            pltpu.make_async_copy(adj_t.at[pl.ds(0, 2)], ip[p3], sI[p3]).wait()

        pltpu.sync_copy(adj_t.at[pl.ds(s * _TJ, 2)], ip[0])
        pltpu.sync_copy(adj_t.at[pl.ds(s * _TJ + 2, 2)], ip[1])
        fire_pair(2, 2)
        xf(0, 0, 0)
        xf(0, 1, 1)
        for jp in (0, 1):
            xf(1, jp, jp + 2)
            wait_g(jp)
            fire_s(jp)

        def main_body(kk, _):
            for u in range(12):
                j = 2 + 12 * kk + u
                rsl = (2 + u) % 4
                rsl2 = u % 4
                wait_s(rsl2)
                if u % 2 == 0:
                    wait_pair(((4 + u) % 6) // 2)

                    @pl.when(j < _TJ - 4)
                    def _():
                        fire_pair((j + 4) // 2, (u % 6) // 2)
                xf(((4 + u) % 6) // 2, u % 2, rsl2)
                wait_g(rsl)
                fire_s(rsl)
            return 0
        lax.fori_loop(0, (_TJ - 4) // 12, main_body, 0)

        for jt in (_TJ - 2, _TJ - 1):
            wait_s((jt - 2) % 4)
            wait_g(jt % 4)
            fire_s(jt % 4)
        wait_s((_TJ - 2) % 4)
        wait_s((_TJ - 1) % 4)
        plsc.subcore_barrier()

        nk = jnp.minimum(_PT // _EC, (_N - s * _PT + _EC - 1) // _EC)

        def ep_chunk(k, _):
            n0 = s * _PT + k * _EC
            pltpu.sync_copy(msg_sp.at[pl.ds(n0, _EC)], mb)
            pltpu.sync_copy(acc4.at[pl.ds(gbase + n0, _EC)], ab)
            pltpu.sync_copy(dinv_h.at[pl.ds(n0, _EC)], dv)
            pltpu.sync_copy(dinv2_h.at[pl.ds(n0, _EC)], d2)

            def grp(q, _):
                dvv = dv[pl.ds(q * 16, 16)]
                d2v = d2[pl.ds(q * 16, 16)]
                for tt in range(16):
                    r = q * 16 + tt
                    m = mb[r, :]
                    tnb[r, :] = m * d2v[tt]
                    mb[r, :] = ab[r, :] + m * dvv[tt]
                return 0
            lax.fori_loop(0, _EC // 16, grp, 0)
            pltpu.sync_copy(mb, accn_o.at[pl.ds(gbase + n0, _EC)])
            pltpu.sync_copy(tnb, tn_o.at[pl.ds(gbase + n0, _EC)])
            return 0
        lax.fori_loop(0, nk, ep_chunk, 0)
        plsc.subcore_barrier()


@functools.partial(
    pl.kernel,
    mesh=_MESH,
    compiler_params=pltpu.CompilerParams(use_tc_tiling_on_sc=False, needs_layout_passes=False),
    out_type=jax.ShapeDtypeStruct((_B,), jnp.float32),
    scratch_types=[
        pltpu.VMEM((128,), jnp.int32),
        pltpu.VMEM((128,), jnp.int32),
        pltpu.VMEM((128,), jnp.int32),
        pltpu.VMEM((128,), jnp.int32),
        pltpu.VMEM((128, _GC), jnp.float32),
        pltpu.VMEM((128, _GC), jnp.float32),
        pltpu.VMEM((128, _GC), jnp.float32),
        pltpu.VMEM((128,), jnp.float32),
        pltpu.SemaphoreType.DMA,
        pltpu.SemaphoreType.DMA,
    ],
)
def _score_kernel(acc4, u_h, i_h, out,
                  uid, iid, gxu, gxi, ru, ri, pb, sb, semU, semI):
    c = lax.axis_index("c")
    s = lax.axis_index("s")
    wid = s * 2 + c
    base = wid * 128
    pltpu.sync_copy(u_h.at[pl.ds(base, 128)], uid)
    pltpu.sync_copy(i_h.at[pl.ds(base, 128)], iid)
    _fill_zero_rows(pb, 128)

    for g in range(_NG):
        for k in range(8):
            sl = pl.ds(k * 16, 16)
            gxu[sl] = uid[sl] + g * _N
            gxi[sl] = iid[sl] + (g * _N + _NU)
        dU = pltpu.async_copy(acc4.at[gxu], ru, semU)
        dI = pltpu.async_copy(acc4.at[gxi], ri, semI)
        dU.wait()
        dI.wait()

        def row(r, _):
            pb[r, :] = pb[r, :] + ru[r, :] * ri[r, :]
            return 0
        lax.fori_loop(0, 128, row, 0)

    lanes = lax.iota(jnp.int32, 16)

    def sgrp(q, _):
        vec = jnp.zeros((16,), jnp.float32)
        for tt in range(16):
            r = q * 16 + tt
            sc = jnp.sum(pb[r, :]) * 0.0625
            vec = jnp.where(lanes == tt, sc, vec)
        sb[pl.ds(q * 16, 16)] = vec
        return 0
    lax.fori_loop(0, 8, sgrp, 0)
    pltpu.sync_copy(sb, out.at[pl.ds(base, 128)])


def kernel(U, V, u, i, adj_user, adj_item):
    e0g = (jnp.concatenate([U, V], axis=0)
           .reshape(_N, _NG, _GC).transpose(1, 0, 2).reshape(_NG * _N, _GC))
    npad = _NS * _TJ * _CK - _NNZ
    au_p = jnp.concatenate([adj_user.astype(jnp.int32),
                            jnp.full((npad,), 100100, jnp.int32)])
    ai_p = jnp.concatenate([adj_item.astype(jnp.int32),
                            jnp.full((npad,), 50100, jnp.int32)])
    adj_t = jnp.stack([au_p.reshape(_NS * _TJ, _CK),
                       ai_p.reshape(_NS * _TJ, _CK)], axis=1)
    dinv, dinv2, t = _prep_kernel(adj_t, e0g)
    acc = e0g
    for _ in range(_NLAYERS):
        t, acc = _layer_kernel(adj_t, t, acc, dinv, dinv2)
    return _score_kernel(acc, u.astype(jnp.int32), i.astype(jnp.int32))


# speedup vs baseline: 1.3041x; 1.3041x over previous
import functools
import jax
import jax.numpy as jnp
from jax import lax
from jax.experimental import pallas as pl
from jax.experimental.pallas import tpu as pltpu
from jax.experimental.pallas import tpu_sc as plsc

_NU = 50000
_NI = 50000
_N = 100000
_NP = 102400
_D = 64
_NG = 4
_GC = 16
_NLAYERS = 3
_B = 4096
_NNZ = 800000
_CK = 128
_NCHUNK = _NNZ // _CK
_NS = 16
_PT = _NP // _NS
_EC = 160
_ZR = 1280
_TJ = 392
_TC = _NS * _TJ

_MESH = plsc.VectorSubcoreMesh(core_axis_name="c", subcore_axis_name="s")
_MAGIC = 0x5F3759DF


def _fill_zero_rows(buf, nrows):
    def body(r, _):
        buf[r, :] = jnp.zeros((16,), jnp.float32)
        return 0
    lax.fori_loop(0, nrows, body, 0)


def _rsqrt16(x):
    bits = plsc.bitcast(x, jnp.int32)
    y = plsc.bitcast(_MAGIC - lax.shift_right_logical(bits, 1), jnp.float32)
    for _ in range(3):
        y = y * (1.5 - 0.5 * x * y * y)
    return y


@functools.partial(
    pl.kernel,
    mesh=_MESH,
    compiler_params=pltpu.CompilerParams(use_tc_tiling_on_sc=False, needs_layout_passes=False),
    out_type=(
        jax.ShapeDtypeStruct((_NP,), jnp.float32),
        jax.ShapeDtypeStruct((_NP,), jnp.float32),
        jax.ShapeDtypeStruct((_NG * _N, _GC), jnp.float32),
    ),
    scratch_types=(
        [pltpu.VMEM_SHARED((_NP,), jnp.float32),
         pltpu.VMEM_SHARED((_NP,), jnp.float32)]
        + [pltpu.VMEM((2, _CK), jnp.int32) for _ in range(4)]
        + [pltpu.VMEM((_CK,), jnp.int32) for _ in range(4)]
        + [pltpu.SemaphoreType.DMA for _ in range(4)]
        + [
            pltpu.VMEM((_CK,), jnp.float32),
            pltpu.VMEM((_ZR,), jnp.float32),
            pltpu.VMEM((_PT,), jnp.float32),
            pltpu.VMEM((_PT,), jnp.float32),
            pltpu.VMEM((_PT,), jnp.float32),
            pltpu.VMEM((1008, _GC), jnp.float32),
            pltpu.VMEM((1008, _GC), jnp.float32),
            pltpu.VMEM((1008,), jnp.float32),
        ]
    ),
)
def _prep_kernel(adj_c, e0g, dinv_o, dinv2_o, t0_o,
                 deg_sp, dinv_sp, ib0, ib1, ib2, ib3,
                 aN0, aN1, aN2, aN3, sS0, sS1, sS2, sS3, ones_v, zb,
                 degb, dvb, d2b, eb, tb, dv1k):
    c = lax.axis_index("c")
    s = lax.axis_index("s")

    @pl.when(c == 0)
    def _():
        def zrow(r, _):
            zb[pl.ds(r * 16, 16)] = jnp.zeros((16,), jnp.float32)
            return 0
        lax.fori_loop(0, _ZR // 16, zrow, 0)
        for k in range(_PT // _ZR):
            pltpu.sync_copy(zb, deg_sp.at[pl.ds(s * _PT + k * _ZR, _ZR)])
        for k in range(_CK // 16):
            ones_v[pl.ds(k * 16, 16)] = jnp.ones((16,), jnp.float32)
        plsc.subcore_barrier()

        ib = (ib0, ib1, ib2, ib3)
        aN = (aN0, aN1, aN2, aN3)
        sS = (sS0, sS1, sS2, sS3)

        def load_x(j, sl):
            pltpu.sync_copy(adj_c.at[s + 16 * j], ib[sl])
            for k in range(_CK // 16):
                ssl = pl.ds(k * 16, 16)
                aN[sl][ssl] = ib[sl][1, ssl] + _NU

        def fire_s(sl):
            pltpu.async_copy(ones_v, deg_sp.at[ib[sl].at[0]], sS[sl], add=True)
            pltpu.async_copy(ones_v, deg_sp.at[aN[sl]], sS[sl], add=True)

        def wait_s(sl):
            pltpu.make_async_copy(ones_v, deg_sp.at[ib[sl].at[0]], sS[sl]).wait()
            pltpu.make_async_copy(ones_v, deg_sp.at[aN[sl]], sS[sl]).wait()

        for u in range(4):
            load_x(u, u)
            fire_s(u)

        def main_body(kk, _):
            for u in range(4):
                wait_s(u)
                load_x(4 + 4 * kk + u, u)
                fire_s(u)
            return 0
        lax.fori_loop(0, (_TJ - 4) // 4, main_body, 0)
        for u in range(4):
            wait_s(u)
        plsc.subcore_barrier()

        base = s * _PT
        pltpu.sync_copy(deg_sp.at[pl.ds(base, _PT)], degb)

        def unit(k, _):
            sl = pl.ds(k * 16, 16)
            v = degb[sl]
            pos = v > 0.0
            x = jnp.maximum(v, 1.0)
            y = _rsqrt16(x)
            dvb[sl] = jnp.where(pos, y, 0.0)
            d2b[sl] = jnp.where(pos, 1.0 / x, 0.0)
            return 0
        lax.fori_loop(0, _PT // 16, unit, 0)
        pltpu.sync_copy(dvb, dinv_o.at[pl.ds(base, _PT)])
        pltpu.sync_copy(d2b, dinv2_o.at[pl.ds(base, _PT)])
        pltpu.sync_copy(dvb, dinv_sp.at[pl.ds(base, _PT)])
        plsc.subcore_barrier()

        def t0_chunk(j, _):
            rb = s * 25000 + j * 1000
            n0 = lax.rem(rb, _N)
            pltpu.sync_copy(e0g.at[pl.ds(rb, 1000)], eb.at[pl.ds(0, 1000)])
            pltpu.sync_copy(dinv_sp.at[pl.ds(n0, 1000)], dv1k.at[pl.ds(0, 1000)])

            def grp(q, _):
                dvv = dv1k[pl.ds(q * 16, 16)]
                for tt in range(16):
                    r = q * 16 + tt
                    tb[r, :] = eb[r, :] * dvv[tt]
                return 0
            lax.fori_loop(0, 1000 // 16 + 1, grp, 0)
            pltpu.sync_copy(tb.at[pl.ds(0, 1000)], t0_o.at[pl.ds(rb, 1000)])
            return 0
        lax.fori_loop(0, 25, t0_chunk, 0)


@functools.partial(
    pl.kernel,
    mesh=_MESH,
    compiler_params=pltpu.CompilerParams(use_tc_tiling_on_sc=False, needs_layout_passes=False),
    out_type=(
        jax.ShapeDtypeStruct((_NG * _N, _GC), jnp.float32),
        jax.ShapeDtypeStruct((_NG * _N, _GC), jnp.float32),
    ),
    scratch_types=(
        [pltpu.VMEM_SHARED((_NP, _GC), jnp.float32)]
        + [pltpu.VMEM((2, _CK), jnp.int32) for _ in range(4)]
        + [pltpu.VMEM((_CK,), jnp.int32) for _ in range(12)]
        + [pltpu.VMEM((_CK, _GC), jnp.float32) for _ in range(8)]
        + [pltpu.SemaphoreType.DMA for _ in range(8)]
        + [
            pltpu.VMEM((_EC, _GC), jnp.float32),
            pltpu.VMEM((_EC, _GC), jnp.float32),
            pltpu.VMEM((_EC, _GC), jnp.float32),
            pltpu.VMEM((_EC,), jnp.float32),
            pltpu.VMEM((_EC,), jnp.float32),
        ]
    ),
)
def _layer_kernel(adj_c, t4, acc4, dinv_h, dinv2_h,
                  tn_o, accn_o, msg_sp,
                  ib0, ib1, ib2, ib3,
                  gA0, gA1, gA2, gA3, gB0, gB1, gB2, gB3,
                  aN0, aN1, aN2, aN3,
                  rA0, rA1, rA2, rA3, rB0, rB1, rB2, rB3,
                  sG0, sG1, sG2, sG3, sS0, sS1, sS2, sS3,
                  mb, ab, tnb, dv, d2):
    c = lax.axis_index("c")
    s = lax.axis_index("s")
    ib = (ib0, ib1, ib2, ib3)
    gA = (gA0, gA1, gA2, gA3)
    gB = (gB0, gB1, gB2, gB3)
    aN = (aN0, aN1, aN2, aN3)
    rA = (rA0, rA1, rA2, rA3)
    rB = (rB0, rB1, rB2, rB3)
    sG = (sG0, sG1, sG2, sG3)
    sS = (sS0, sS1, sS2, sS3)
    clamp = _NG * _N - 1

    for p in range(2):
        g = c * 2 + p
        gbase = g * _N

        _fill_zero_rows(rA0, _CK)

        def zcopy(k, _):
            pltpu.sync_copy(rA0, msg_sp.at[pl.ds(s * _PT + k * _CK, _CK)])
            return 0
        lax.fori_loop(0, _PT // _CK, zcopy, 0)
        plsc.subcore_barrier()

        def load_x(j, sl):
            cidx = s + 16 * j
            pltpu.sync_copy(adj_c.at[cidx], ib[sl])
            for k in range(_CK // 16):
                ssl = pl.ds(k * 16, 16)
                av = ib[sl][0, ssl]
                iv = ib[sl][1, ssl]
                gA[sl][ssl] = jnp.minimum(av + gbase, clamp)
                aN[sl][ssl] = iv + _NU
                gB[sl][ssl] = jnp.minimum(iv + (gbase + _NU), clamp)
            pltpu.async_copy(t4.at[gA[sl]], rA[sl], sG[sl])
            pltpu.async_copy(t4.at[gB[sl]], rB[sl], sG[sl])

        def wait_g(sl):
            pltpu.make_async_copy(t4.at[gA[sl]], rA[sl], sG[sl]).wait()
            pltpu.make_async_copy(t4.at[gB[sl]], rB[sl], sG[sl]).wait()

        def fire_s(sl):
            pltpu.async_copy(rA[sl], msg_sp.at[aN[sl]], sS[sl], add=True)
            pltpu.async_copy(rB[sl], msg_sp.at[ib[sl].at[0]], sS[sl], add=True)

        def wait_s(sl):
            pltpu.make_async_copy(rA[sl], msg_sp.at[aN[sl]], sS[sl]).wait()
            pltpu.make_async_copy(rB[sl], msg_sp.at[ib[sl].at[0]], sS[sl]).wait()

        load_x(0, 0)
        load_x(1, 1)
        for u in range(2):
            load_x(u + 2, (u + 2) % 4)
            wait_g(u)
            fire_s(u)

        def main_body(kk, _):
            for u in range(4):
                j = 2 + 4 * kk + u
                sl = (2 + u) % 4
                sl2 = u
                wait_s(sl2)
                load_x(j + 2, sl2)
                wait_g(sl)
                fire_s(sl)
            return 0
        lax.fori_loop(0, (_TJ - 4) // 4, main_body, 0)

        for jt in (_TJ - 2, _TJ - 1):
            sl = jt % 4
            wait_s((jt - 2) % 4)
            wait_g(sl)
            fire_s(sl)
        wait_s((_TJ - 2) % 4)
        wait_s((_TJ - 1) % 4)
        plsc.subcore_barrier()

        nk = jnp.minimum(_PT // _EC, (_N - s * _PT + _EC - 1) // _EC)

        def ep_chunk(k, _):
            n0 = s * _PT + k * _EC
            pltpu.sync_copy(msg_sp.at[pl.ds(n0, _EC)], mb)
            pltpu.sync_copy(acc4.at[pl.ds(gbase + n0, _EC)], ab)
            pltpu.sync_copy(dinv_h.at[pl.ds(n0, _EC)], dv)
            pltpu.sync_copy(dinv2_h.at[pl.ds(n0, _EC)], d2)

            def grp(q, _):
                dvv = dv[pl.ds(q * 16, 16)]
                d2v = d2[pl.ds(q * 16, 16)]
                for tt in range(16):
                    r = q * 16 + tt
                    m = mb[r, :]
                    tnb[r, :] = m * d2v[tt]
                    mb[r, :] = ab[r, :] + m * dvv[tt]
                return 0
            lax.fori_loop(0, _EC // 16, grp, 0)
            pltpu.sync_copy(mb, accn_o.at[pl.ds(gbase + n0, _EC)])
            pltpu.sync_copy(tnb, tn_o.at[pl.ds(gbase + n0, _EC)])
            return 0
        lax.fori_loop(0, nk, ep_chunk, 0)
        plsc.subcore_barrier()


@functools.partial(
    pl.kernel,
    mesh=_MESH,
    compiler_params=pltpu.CompilerParams(use_tc_tiling_on_sc=False, needs_layout_passes=False),
    out_type=jax.ShapeDtypeStruct((_B,), jnp.float32),
    scratch_types=[
        pltpu.VMEM((128,), jnp.int32),
        pltpu.VMEM((128,), jnp.int32),
        pltpu.VMEM((128,), jnp.int32),
        pltpu.VMEM((128,), jnp.int32),
        pltpu.VMEM((128, _GC), jnp.float32),
        pltpu.VMEM((128, _GC), jnp.float32),
        pltpu.VMEM((128, _GC), jnp.float32),
        pltpu.VMEM((128,), jnp.float32),
        pltpu.SemaphoreType.DMA,
        pltpu.SemaphoreType.DMA,
    ],
)
def _score_kernel(acc4, u_h, i_h, out,
                  uid, iid, gxu, gxi, ru, ri, pb, sb, semU, semI):
    c = lax.axis_index("c")
    s = lax.axis_index("s")
    wid = s * 2 + c
    base = wid * 128
    pltpu.sync_copy(u_h.at[pl.ds(base, 128)], uid)
    pltpu.sync_copy(i_h.at[pl.ds(base, 128)], iid)
    _fill_zero_rows(pb, 128)

    for g in range(_NG):
        for k in range(8):
            sl = pl.ds(k * 16, 16)
            gxu[sl] = uid[sl] + g * _N
            gxi[sl] = iid[sl] + (g * _N + _NU)
        dU = pltpu.async_copy(acc4.at[gxu], ru, semU)
        dI = pltpu.async_copy(acc4.at[gxi], ri, semI)
        dU.wait()
        dI.wait()

        def row(r, _):
            pb[r, :] = pb[r, :] + ru[r, :] * ri[r, :]
            return 0
        lax.fori_loop(0, 128, row, 0)

    lanes = lax.iota(jnp.int32, 16)

    def sgrp(q, _):
        vec = jnp.zeros((16,), jnp.float32)
        for tt in range(16):
            r = q * 16 + tt
            sc = jnp.sum(pb[r, :]) * 0.0625
            vec = jnp.where(lanes == tt, sc, vec)
        sb[pl.ds(q * 16, 16)] = vec
        return 0
    lax.fori_loop(0, 8, sgrp, 0)
    pltpu.sync_copy(sb, out.at[pl.ds(base, 128)])


def kernel(U, V, u, i, adj_user, adj_item):
    e0g = (jnp.concatenate([U, V], axis=0)
           .reshape(_N, _NG, _GC).transpose(1, 0, 2).reshape(_NG * _N, _GC))
    npad = _TC * _CK - _NNZ
    au_p = jnp.concatenate([adj_user.astype(jnp.int32),
                            jnp.full((npad,), 101000, jnp.int32)])
    ai_p = jnp.concatenate([adj_item.astype(jnp.int32),
                            jnp.full((npad,), 51000, jnp.int32)])
    adj_c = jnp.stack([au_p.reshape(_TC, _CK), ai_p.reshape(_TC, _CK)], axis=1)
    dinv, dinv2, t = _prep_kernel(adj_c, e0g)
    acc = e0g
    for _ in range(_NLAYERS):
        t, acc = _layer_kernel(adj_c, t, acc, dinv, dinv2)
    return _score_kernel(acc, u.astype(jnp.int32), i.astype(jnp.int32))


# speedup vs baseline: 1.4429x; 1.1064x over previous
import functools
import jax
import jax.numpy as jnp
from jax import lax
from jax.experimental import pallas as pl
from jax.experimental.pallas import tpu as pltpu
from jax.experimental.pallas import tpu_sc as plsc

_NU = 50000
_NI = 50000
_N = 100000
_NP = 102400
_D = 64
_NG = 4
_GC = 16
_NLAYERS = 3
_B = 4096
_NNZ = 800000
_CK = 128
_NCHUNK = _NNZ // _CK
_NS = 16
_PT = _NP // _NS
_EC = 160
_ZR = 1280
_TJ = 392
_TC = _NS * _TJ

_MESH = plsc.VectorSubcoreMesh(core_axis_name="c", subcore_axis_name="s")
_MAGIC = 0x5F3759DF


def _fill_zero_rows(buf, nrows):
    def body(r, _):
        buf[r, :] = jnp.zeros((16,), jnp.float32)
        return 0
    lax.fori_loop(0, nrows, body, 0)


def _rsqrt16(x):
    bits = plsc.bitcast(x, jnp.int32)
    y = plsc.bitcast(_MAGIC - lax.shift_right_logical(bits, 1), jnp.float32)
    for _ in range(3):
        y = y * (1.5 - 0.5 * x * y * y)
    return y


@functools.partial(
    pl.kernel,
    mesh=_MESH,
    compiler_params=pltpu.CompilerParams(use_tc_tiling_on_sc=False, needs_layout_passes=False),
    out_type=(
        jax.ShapeDtypeStruct((_NP,), jnp.float32),
        jax.ShapeDtypeStruct((_NP,), jnp.float32),
        jax.ShapeDtypeStruct((_NG * _N, _GC), jnp.float32),
    ),
    scratch_types=(
        [pltpu.VMEM_SHARED((_NP,), jnp.float32),
         pltpu.VMEM_SHARED((_NP,), jnp.float32)]
        + [pltpu.VMEM((2, _CK), jnp.int32) for _ in range(4)]
        + [pltpu.VMEM((_CK,), jnp.int32) for _ in range(4)]
        + [pltpu.SemaphoreType.DMA for _ in range(4)]
        + [
            pltpu.VMEM((_CK,), jnp.float32),
            pltpu.VMEM((_ZR,), jnp.float32),
            pltpu.VMEM((_PT,), jnp.float32),
            pltpu.VMEM((_PT,), jnp.float32),
            pltpu.VMEM((_PT,), jnp.float32),
            pltpu.VMEM((1008, _GC), jnp.float32),
            pltpu.VMEM((1008, _GC), jnp.float32),
            pltpu.VMEM((1008,), jnp.float32),
        ]
    ),
)
def _prep_kernel(adj_c, e0g, dinv_o, dinv2_o, t0_o,
                 deg_sp, dinv_sp, ib0, ib1, ib2, ib3,
                 aN0, aN1, aN2, aN3, sS0, sS1, sS2, sS3, ones_v, zb,
                 degb, dvb, d2b, eb, tb, dv1k):
    c = lax.axis_index("c")
    s = lax.axis_index("s")

    @pl.when(c == 0)
    def _():
        def zrow(r, _):
            zb[pl.ds(r * 16, 16)] = jnp.zeros((16,), jnp.float32)
            return 0
        lax.fori_loop(0, _ZR // 16, zrow, 0)
        for k in range(_PT // _ZR):
            pltpu.sync_copy(zb, deg_sp.at[pl.ds(s * _PT + k * _ZR, _ZR)])
        for k in range(_CK // 16):
            ones_v[pl.ds(k * 16, 16)] = jnp.ones((16,), jnp.float32)
        plsc.subcore_barrier()

        ib = (ib0, ib1, ib2, ib3)
        aN = (aN0, aN1, aN2, aN3)
        sS = (sS0, sS1, sS2, sS3)

        def load_x(j, sl):
            pltpu.sync_copy(adj_c.at[s + 16 * j], ib[sl])
            for k in range(_CK // 16):
                ssl = pl.ds(k * 16, 16)
                aN[sl][ssl] = ib[sl][1, ssl] + _NU

        def fire_s(sl):
            pltpu.async_copy(ones_v, deg_sp.at[ib[sl].at[0]], sS[sl], add=True)
            pltpu.async_copy(ones_v, deg_sp.at[aN[sl]], sS[sl], add=True)

        def wait_s(sl):
            pltpu.make_async_copy(ones_v, deg_sp.at[ib[sl].at[0]], sS[sl]).wait()
            pltpu.make_async_copy(ones_v, deg_sp.at[aN[sl]], sS[sl]).wait()

        for u in range(4):
            load_x(u, u)
            fire_s(u)

        def main_body(kk, _):
            for u in range(4):
                wait_s(u)
                load_x(4 + 4 * kk + u, u)
                fire_s(u)
            return 0
        lax.fori_loop(0, (_TJ - 4) // 4, main_body, 0)
        for u in range(4):
            wait_s(u)
        plsc.subcore_barrier()

        base = s * _PT
        pltpu.sync_copy(deg_sp.at[pl.ds(base, _PT)], degb)

        def unit(k, _):
            sl = pl.ds(k * 16, 16)
            v = degb[sl]
            pos = v > 0.0
            x = jnp.maximum(v, 1.0)
            y = _rsqrt16(x)
            dvb[sl] = jnp.where(pos, y, 0.0)
            d2b[sl] = jnp.where(pos, 1.0 / x, 0.0)
            return 0
        lax.fori_loop(0, _PT // 16, unit, 0)
        pltpu.sync_copy(dvb, dinv_o.at[pl.ds(base, _PT)])
        pltpu.sync_copy(d2b, dinv2_o.at[pl.ds(base, _PT)])
        pltpu.sync_copy(dvb, dinv_sp.at[pl.ds(base, _PT)])
        plsc.subcore_barrier()

        def t0_chunk(j, _):
            rb = s * 25000 + j * 1000
            n0 = lax.rem(rb, _N)
            pltpu.sync_copy(e0g.at[pl.ds(rb, 1000)], eb.at[pl.ds(0, 1000)])
            pltpu.sync_copy(dinv_sp.at[pl.ds(n0, 1000)], dv1k.at[pl.ds(0, 1000)])

            def grp(q, _):
                dvv = dv1k[pl.ds(q * 16, 16)]
                for tt in range(16):
                    r = q * 16 + tt
                    tb[r, :] = eb[r, :] * dvv[tt]
                return 0
            lax.fori_loop(0, 1000 // 16 + 1, grp, 0)
            pltpu.sync_copy(tb.at[pl.ds(0, 1000)], t0_o.at[pl.ds(rb, 1000)])
            return 0
        lax.fori_loop(0, 25, t0_chunk, 0)


@functools.partial(
    pl.kernel,
    mesh=_MESH,
    compiler_params=pltpu.CompilerParams(use_tc_tiling_on_sc=False, needs_layout_passes=False),
    out_type=(
        jax.ShapeDtypeStruct((_NG * _N, _GC), jnp.float32),
        jax.ShapeDtypeStruct((_NG * _N, _GC), jnp.float32),
    ),
    scratch_types=(
        [pltpu.VMEM_SHARED((_NP, _GC), jnp.float32)]
        + [pltpu.VMEM((2, _CK), jnp.int32) for _ in range(4)]
        + [pltpu.VMEM((_CK,), jnp.int32) for _ in range(12)]
        + [pltpu.VMEM((_CK, _GC), jnp.float32) for _ in range(8)]
        + [pltpu.SemaphoreType.DMA for _ in range(8)]
        + [
            pltpu.VMEM((_EC, _GC), jnp.float32),
            pltpu.VMEM((_EC, _GC), jnp.float32),
            pltpu.VMEM((_EC, _GC), jnp.float32),
            pltpu.VMEM((_EC,), jnp.float32),
            pltpu.VMEM((_EC,), jnp.float32),
        ]
    ),
)
def _layer_kernel(adj_c, t4, acc4, dinv_h, dinv2_h,
                  tn_o, accn_o, msg_sp,
                  ib0, ib1, ib2, ib3,
                  gA0, gA1, gA2, gA3, gB0, gB1, gB2, gB3,
                  aN0, aN1, aN2, aN3,
                  rA0, rA1, rA2, rA3, rB0, rB1, rB2, rB3,
                  sG0, sG1, sG2, sG3, sS0, sS1, sS2, sS3,
                  mb, ab, tnb, dv, d2):
    c = lax.axis_index("c")
    s = lax.axis_index("s")
    ib = (ib0, ib1, ib2, ib3)
    gA = (gA0, gA1, gA2, gA3)
    gB = (gB0, gB1, gB2, gB3)
    aN = (aN0, aN1, aN2, aN3)
    rA = (rA0, rA1, rA2, rA3)
    rB = (rB0, rB1, rB2, rB3)
    sG = (sG0, sG1, sG2, sG3)
    sS = (sS0, sS1, sS2, sS3)
    clamp = _NG * _N - 1

    for p in range(2):
        g = c * 2 + p
        gbase = g * _N

        _fill_zero_rows(rA0, _CK)

        def zcopy(k, _):
            pltpu.sync_copy(rA0, msg_sp.at[pl.ds(s * _PT + k * _CK, _CK)])
            return 0
        lax.fori_loop(0, _PT // _CK, zcopy, 0)
        plsc.subcore_barrier()

        def load_x(j, sl):
            cidx = s + 16 * j
            pltpu.sync_copy(adj_c.at[cidx], ib[sl])
            for k in range(_CK // 16):
                ssl = pl.ds(k * 16, 16)
                av = ib[sl][0, ssl]
                iv = ib[sl][1, ssl]
                gA[sl][ssl] = jnp.minimum(av + gbase, clamp)
                aN[sl][ssl] = iv + _NU
                gB[sl][ssl] = jnp.minimum(iv + (gbase + _NU), clamp)
            pltpu.async_copy(t4.at[gA[sl]], rA[sl], sG[sl])
            pltpu.async_copy(t4.at[gB[sl]], rB[sl], sG[sl])

        def wait_g(sl):
            pltpu.make_async_copy(t4.at[gA[sl]], rA[sl], sG[sl]).wait()
            pltpu.make_async_copy(t4.at[gB[sl]], rB[sl], sG[sl]).wait()

        def fire_s(sl):
            pltpu.async_copy(rA[sl], msg_sp.at[aN[sl]], sS[sl], add=True)
            pltpu.async_copy(rB[sl], msg_sp.at[ib[sl].at[0]], sS[sl], add=True)

        def wait_s(sl):
            pltpu.make_async_copy(rA[sl], msg_sp.at[aN[sl]], sS[sl]).wait()
            pltpu.make_async_copy(rB[sl], msg_sp.at[ib[sl].at[0]], sS[sl]).wait()

        load_x(0, 0)
        load_x(1, 1)
        for u in range(2):
            load_x(u + 2, (u + 2) % 4)
            wait_g(u)
            fire_s(u)

        def main_body(kk, _):
            for u in range(4):
                j = 2 + 4 * kk + u
                sl = (2 + u) % 4
                sl2 = u
                wait_s(sl2)
                load_x(j + 2, sl2)
                wait_g(sl)
                fire_s(sl)
            return 0
        lax.fori_loop(0, (_TJ - 4) // 4, main_body, 0)

        for jt in (_TJ - 2, _TJ - 1):
            sl = jt % 4
            wait_s((jt - 2) % 4)
            wait_g(sl)
            fire_s(sl)
        wait_s((_TJ - 2) % 4)
        wait_s((_TJ - 1) % 4)
        plsc.subcore_barrier()

        nk = jnp.minimum(_PT // _EC, (_N - s * _PT + _EC - 1) // _EC)

        def ep_chunk(k, _):
            n0 = s * _PT + k * _EC
            e1 = pltpu.async_copy(msg_sp.at[pl.ds(n0, _EC)], mb, sG0)
            e2 = pltpu.async_copy(acc4.at[pl.ds(gbase + n0, _EC)], ab, sG1)
            e3 = pltpu.async_copy(dinv_h.at[pl.ds(n0, _EC)], dv, sG2)
            e4 = pltpu.async_copy(dinv2_h.at[pl.ds(n0, _EC)], d2, sG3)
            e1.wait()
            e2.wait()
            e3.wait()
            e4.wait()

            def grp(q, _):
                dvv = dv[pl.ds(q * 16, 16)]
                d2v = d2[pl.ds(q * 16, 16)]
                for tt in range(16):
                    r = q * 16 + tt
                    m = mb[r, :]
                    tnb[r, :] = m * d2v[tt]
                    mb[r, :] = ab[r, :] + m * dvv[tt]
                return 0
            lax.fori_loop(0, _EC // 16, grp, 0)
            pltpu.sync_copy(mb, accn_o.at[pl.ds(gbase + n0, _EC)])
            pltpu.sync_copy(tnb, tn_o.at[pl.ds(gbase + n0, _EC)])
            return 0
        lax.fori_loop(0, nk, ep_chunk, 0)
        plsc.subcore_barrier()


@functools.partial(
    pl.kernel,
    mesh=_MESH,
    compiler_params=pltpu.CompilerParams(use_tc_tiling_on_sc=False, needs_layout_passes=False),
    out_type=jax.ShapeDtypeStruct((_B,), jnp.float32),
    scratch_types=[
        pltpu.VMEM((128,), jnp.int32),
        pltpu.VMEM((128,), jnp.int32),
        pltpu.VMEM((128,), jnp.int32),
        pltpu.VMEM((128,), jnp.int32),
        pltpu.VMEM((128, _GC), jnp.float32),
        pltpu.VMEM((128, _GC), jnp.float32),
        pltpu.VMEM((128, _GC), jnp.float32),
        pltpu.VMEM((128,), jnp.float32),
        pltpu.SemaphoreType.DMA,
        pltpu.SemaphoreType.DMA,
    ],
)
def _score_kernel(acc4, u_h, i_h, out,
                  uid, iid, gxu, gxi, ru, ri, pb, sb, semU, semI):
    c = lax.axis_index("c")
    s = lax.axis_index("s")
    wid = s * 2 + c
    base = wid * 128
    pltpu.sync_copy(u_h.at[pl.ds(base, 128)], uid)
    pltpu.sync_copy(i_h.at[pl.ds(base, 128)], iid)
    _fill_zero_rows(pb, 128)

    for g in range(_NG):
        for k in range(8):
            sl = pl.ds(k * 16, 16)
            gxu[sl] = uid[sl] + g * _N
            gxi[sl] = iid[sl] + (g * _N + _NU)
        dU = pltpu.async_copy(acc4.at[gxu], ru, semU)
        dI = pltpu.async_copy(acc4.at[gxi], ri, semI)
        dU.wait()
        dI.wait()

        def row(r, _):
            pb[r, :] = pb[r, :] + ru[r, :] * ri[r, :]
            return 0
        lax.fori_loop(0, 128, row, 0)

    lanes = lax.iota(jnp.int32, 16)

    def sgrp(q, _):
        vec = jnp.zeros((16,), jnp.float32)
        for tt in range(16):
            r = q * 16 + tt
            sc = jnp.sum(pb[r, :]) * 0.0625
            vec = jnp.where(lanes == tt, sc, vec)
        sb[pl.ds(q * 16, 16)] = vec
        return 0
    lax.fori_loop(0, 8, sgrp, 0)
    pltpu.sync_copy(sb, out.at[pl.ds(base, 128)])


def kernel(U, V, u, i, adj_user, adj_item):
    e0g = (jnp.concatenate([U, V], axis=0)
           .reshape(_N, _NG, _GC).transpose(1, 0, 2).reshape(_NG * _N, _GC))
    npad = _TC * _CK - _NNZ
    au_p = jnp.concatenate([adj_user.astype(jnp.int32),
                            jnp.full((npad,), 101000, jnp.int32)])
    ai_p = jnp.concatenate([adj_item.astype(jnp.int32),
                            jnp.full((npad,), 51000, jnp.int32)])
    adj_c = jnp.stack([au_p.reshape(_TC, _CK), ai_p.reshape(_TC, _CK)], axis=1)
    dinv, dinv2, t = _prep_kernel(adj_c, e0g)
    acc = e0g
    for _ in range(_NLAYERS):
        t, acc = _layer_kernel(adj_c, t, acc, dinv, dinv2)
    return _score_kernel(acc, u.astype(jnp.int32), i.astype(jnp.int32))


# speedup vs baseline: 1.4449x; 1.0014x over previous
import functools
import jax
import jax.numpy as jnp
from jax import lax
from jax.experimental import pallas as pl
from jax.experimental.pallas import tpu as pltpu
from jax.experimental.pallas import tpu_sc as plsc

_NU = 50000
_NI = 50000
_N = 100000
_NP = 102400
_D = 64
_NG = 4
_GC = 16
_NLAYERS = 3
_B = 4096
_NNZ = 800000
_CK = 128
_NCHUNK = _NNZ // _CK
_NS = 16
_PT = _NP // _NS
_EC = 160
_ZR = 1280
_TJ = 392
_TC = _NS * _TJ

_MESH = plsc.VectorSubcoreMesh(core_axis_name="c", subcore_axis_name="s")
_MAGIC = 0x5F3759DF


def _fill_zero_rows(buf, nrows):
    def body(r, _):
        buf[r, :] = jnp.zeros((16,), jnp.float32)
        return 0
    lax.fori_loop(0, nrows, body, 0)


def _rsqrt16(x):
    bits = plsc.bitcast(x, jnp.int32)
    y = plsc.bitcast(_MAGIC - lax.shift_right_logical(bits, 1), jnp.float32)
    for _ in range(3):
        y = y * (1.5 - 0.5 * x * y * y)
    return y


@functools.partial(
    pl.kernel,
    mesh=_MESH,
    compiler_params=pltpu.CompilerParams(use_tc_tiling_on_sc=False, needs_layout_passes=False),
    out_type=(
        jax.ShapeDtypeStruct((_NP,), jnp.float32),
        jax.ShapeDtypeStruct((_NP,), jnp.float32),
        jax.ShapeDtypeStruct((_NG * _N, _GC), jnp.float32),
    ),
    scratch_types=(
        [pltpu.VMEM_SHARED((_NP,), jnp.float32),
         pltpu.VMEM_SHARED((_NP,), jnp.float32)]
        + [pltpu.VMEM((2, _CK), jnp.int32) for _ in range(4)]
        + [pltpu.VMEM((_CK,), jnp.int32) for _ in range(4)]
        + [pltpu.SemaphoreType.DMA for _ in range(4)]
        + [
            pltpu.VMEM((_CK,), jnp.float32),
            pltpu.VMEM((_ZR,), jnp.float32),
            pltpu.VMEM((_PT,), jnp.float32),
            pltpu.VMEM((_PT,), jnp.float32),
            pltpu.VMEM((_PT,), jnp.float32),
            pltpu.VMEM((1008, _GC), jnp.float32),
            pltpu.VMEM((1008, _GC), jnp.float32),
            pltpu.VMEM((1008,), jnp.float32),
        ]
    ),
)
def _prep_kernel(adj_c, e0g, dinv_o, dinv2_o, t0_o,
                 deg_sp, dinv_sp, ib0, ib1, ib2, ib3,
                 aN0, aN1, aN2, aN3, sS0, sS1, sS2, sS3, ones_v, zb,
                 degb, dvb, d2b, eb, tb, dv1k):
    c = lax.axis_index("c")
    s = lax.axis_index("s")

    @pl.when(c == 0)
    def _():
        def zrow(r, _):
            zb[pl.ds(r * 16, 16)] = jnp.zeros((16,), jnp.float32)
            return 0
        lax.fori_loop(0, _ZR // 16, zrow, 0)
        for k in range(_PT // _ZR):
            pltpu.sync_copy(zb, deg_sp.at[pl.ds(s * _PT + k * _ZR, _ZR)])
        for k in range(_CK // 16):
            ones_v[pl.ds(k * 16, 16)] = jnp.ones((16,), jnp.float32)
        plsc.subcore_barrier()

        ib = (ib0, ib1, ib2, ib3)
        aN = (aN0, aN1, aN2, aN3)
        sS = (sS0, sS1, sS2, sS3)

        def load_x(j, sl):
            pltpu.sync_copy(adj_c.at[s + 16 * j], ib[sl])
            for k in range(_CK // 16):
                ssl = pl.ds(k * 16, 16)
                aN[sl][ssl] = ib[sl][1, ssl] + _NU

        def fire_s(sl):
            pltpu.async_copy(ones_v, deg_sp.at[ib[sl].at[0]], sS[sl], add=True)
            pltpu.async_copy(ones_v, deg_sp.at[aN[sl]], sS[sl], add=True)

        def wait_s(sl):
            pltpu.make_async_copy(ones_v, deg_sp.at[ib[sl].at[0]], sS[sl]).wait()
            pltpu.make_async_copy(ones_v, deg_sp.at[aN[sl]], sS[sl]).wait()

        for u in range(4):
            load_x(u, u)
            fire_s(u)

        def main_body(kk, _):
            for u in range(4):
                wait_s(u)
                load_x(4 + 4 * kk + u, u)
                fire_s(u)
            return 0
        lax.fori_loop(0, (_TJ - 4) // 4, main_body, 0)
        for u in range(4):
            wait_s(u)
        plsc.subcore_barrier()

        base = s * _PT
        pltpu.sync_copy(deg_sp.at[pl.ds(base, _PT)], degb)

        def unit(k, _):
            sl = pl.ds(k * 16, 16)
            v = degb[sl]
            pos = v > 0.0
            x = jnp.maximum(v, 1.0)
            y = _rsqrt16(x)
            dvb[sl] = jnp.where(pos, y, 0.0)
            d2b[sl] = jnp.where(pos, 1.0 / x, 0.0)
            return 0
        lax.fori_loop(0, _PT // 16, unit, 0)
        pltpu.sync_copy(dvb, dinv_o.at[pl.ds(base, _PT)])
        pltpu.sync_copy(d2b, dinv2_o.at[pl.ds(base, _PT)])
        pltpu.sync_copy(dvb, dinv_sp.at[pl.ds(base, _PT)])
        plsc.subcore_barrier()

        def t0_chunk(j, _):
            rb = s * 25000 + j * 1000
            n0 = lax.rem(rb, _N)
            e1 = pltpu.async_copy(e0g.at[pl.ds(rb, 1000)],
                                  eb.at[pl.ds(0, 1000)], sS0)
            e2 = pltpu.async_copy(dinv_sp.at[pl.ds(n0, 1000)],
                                  dv1k.at[pl.ds(0, 1000)], sS1)
            e1.wait()
            e2.wait()

            def grp(q, _):
                dvv = dv1k[pl.ds(q * 16, 16)]
                for tt in range(16):
                    r = q * 16 + tt
                    tb[r, :] = eb[r, :] * dvv[tt]
                return 0
            lax.fori_loop(0, 1000 // 16 + 1, grp, 0)
            pltpu.sync_copy(tb.at[pl.ds(0, 1000)], t0_o.at[pl.ds(rb, 1000)])
            return 0
        lax.fori_loop(0, 25, t0_chunk, 0)


@functools.partial(
    pl.kernel,
    mesh=_MESH,
    compiler_params=pltpu.CompilerParams(use_tc_tiling_on_sc=False, needs_layout_passes=False),
    out_type=(
        jax.ShapeDtypeStruct((_NG * _N, _GC), jnp.float32),
        jax.ShapeDtypeStruct((_NG * _N, _GC), jnp.float32),
    ),
    scratch_types=(
        [pltpu.VMEM_SHARED((_NP, _GC), jnp.float32)]
        + [pltpu.VMEM((2, _CK), jnp.int32) for _ in range(4)]
        + [pltpu.VMEM((_CK,), jnp.int32) for _ in range(12)]
        + [pltpu.VMEM((_CK, _GC), jnp.float32) for _ in range(8)]
        + [pltpu.SemaphoreType.DMA for _ in range(8)]
        + [
            pltpu.VMEM((_EC, _GC), jnp.float32),
            pltpu.VMEM((_EC, _GC), jnp.float32),
            pltpu.VMEM((_EC, _GC), jnp.float32),
            pltpu.VMEM((_EC,), jnp.float32),
            pltpu.VMEM((_EC,), jnp.float32),
        ]
    ),
)
def _layer_kernel(adj_c, t4, acc4, dinv_h, dinv2_h,
                  tn_o, accn_o, msg_sp,
                  ib0, ib1, ib2, ib3,
                  gA0, gA1, gA2, gA3, gB0, gB1, gB2, gB3,
                  aN0, aN1, aN2, aN3,
                  rA0, rA1, rA2, rA3, rB0, rB1, rB2, rB3,
                  sG0, sG1, sG2, sG3, sS0, sS1, sS2, sS3,
                  mb, ab, tnb, dv, d2):
    c = lax.axis_index("c")
    s = lax.axis_index("s")
    ib = (ib0, ib1, ib2, ib3)
    gA = (gA0, gA1, gA2, gA3)
    gB = (gB0, gB1, gB2, gB3)
    aN = (aN0, aN1, aN2, aN3)
    rA = (rA0, rA1, rA2, rA3)
    rB = (rB0, rB1, rB2, rB3)
    sG = (sG0, sG1, sG2, sG3)
    sS = (sS0, sS1, sS2, sS3)
    clamp = _NG * _N - 1

    for p in range(2):
        g = c * 2 + p
        gbase = g * _N

        _fill_zero_rows(rA0, _CK)

        def zcopy(k, _):
            pltpu.sync_copy(rA0, msg_sp.at[pl.ds(s * _PT + k * _CK, _CK)])
            return 0
        lax.fori_loop(0, _PT // _CK, zcopy, 0)
        plsc.subcore_barrier()

        def load_x(j, sl):
            cidx = s + 16 * j
            pltpu.sync_copy(adj_c.at[cidx], ib[sl])
            for k in range(_CK // 16):
                ssl = pl.ds(k * 16, 16)
                av = ib[sl][0, ssl]
                iv = ib[sl][1, ssl]
                gA[sl][ssl] = jnp.minimum(av + gbase, clamp)
                aN[sl][ssl] = iv + _NU
                gB[sl][ssl] = jnp.minimum(iv + (gbase + _NU), clamp)
            pltpu.async_copy(t4.at[gA[sl]], rA[sl], sG[sl])
            pltpu.async_copy(t4.at[gB[sl]], rB[sl], sG[sl])

        def wait_g(sl):
            pltpu.make_async_copy(t4.at[gA[sl]], rA[sl], sG[sl]).wait()
            pltpu.make_async_copy(t4.at[gB[sl]], rB[sl], sG[sl]).wait()

        def fire_s(sl):
            pltpu.async_copy(rA[sl], msg_sp.at[aN[sl]], sS[sl], add=True)
            pltpu.async_copy(rB[sl], msg_sp.at[ib[sl].at[0]], sS[sl], add=True)

        def wait_s(sl):
            pltpu.make_async_copy(rA[sl], msg_sp.at[aN[sl]], sS[sl]).wait()
            pltpu.make_async_copy(rB[sl], msg_sp.at[ib[sl].at[0]], sS[sl]).wait()

        load_x(0, 0)
        load_x(1, 1)
        for u in range(2):
            load_x(u + 2, (u + 2) % 4)
            wait_g(u)
            fire_s(u)

        def main_body(kk, _):
            for u in range(4):
                j = 2 + 4 * kk + u
                sl = (2 + u) % 4
                sl2 = u
                wait_s(sl2)
                load_x(j + 2, sl2)
                wait_g(sl)
                fire_s(sl)
            return 0
        lax.fori_loop(0, (_TJ - 4) // 4, main_body, 0)

        for jt in (_TJ - 2, _TJ - 1):
            sl = jt % 4
            wait_s((jt - 2) % 4)
            wait_g(sl)
            fire_s(sl)
        wait_s((_TJ - 2) % 4)
        wait_s((_TJ - 1) % 4)
        plsc.subcore_barrier()

        nk = jnp.minimum(_PT // _EC, (_N - s * _PT + _EC - 1) // _EC)

        def ep_chunk(k, _):
            n0 = s * _PT + k * _EC
            e1 = pltpu.async_copy(msg_sp.at[pl.ds(n0, _EC)], mb, sG0)
            e2 = pltpu.async_copy(acc4.at[pl.ds(gbase + n0, _EC)], ab, sG1)
            e3 = pltpu.async_copy(dinv_h.at[pl.ds(n0, _EC)], dv, sG2)
            e4 = pltpu.async_copy(dinv2_h.at[pl.ds(n0, _EC)], d2, sG3)
            e1.wait()
            e2.wait()
            e3.wait()
            e4.wait()

            def grp(q, _):
                dvv = dv[pl.ds(q * 16, 16)]
                d2v = d2[pl.ds(q * 16, 16)]
                for tt in range(16):
                    r = q * 16 + tt
                    m = mb[r, :]
                    tnb[r, :] = m * d2v[tt]
                    mb[r, :] = ab[r, :] + m * dvv[tt]
                return 0
            lax.fori_loop(0, _EC // 16, grp, 0)
            pltpu.sync_copy(mb, accn_o.at[pl.ds(gbase + n0, _EC)])
            pltpu.sync_copy(tnb, tn_o.at[pl.ds(gbase + n0, _EC)])
            return 0
        lax.fori_loop(0, nk, ep_chunk, 0)
        plsc.subcore_barrier()


@functools.partial(
    pl.kernel,
    mesh=_MESH,
    compiler_params=pltpu.CompilerParams(use_tc_tiling_on_sc=False, needs_layout_passes=False),
    out_type=jax.ShapeDtypeStruct((_B,), jnp.float32),
    scratch_types=[
        pltpu.VMEM((128,), jnp.int32),
        pltpu.VMEM((128,), jnp.int32),
        pltpu.VMEM((128,), jnp.int32),
        pltpu.VMEM((128,), jnp.int32),
        pltpu.VMEM((128, _GC), jnp.float32),
        pltpu.VMEM((128, _GC), jnp.float32),
        pltpu.VMEM((128, _GC), jnp.float32),
        pltpu.VMEM((128,), jnp.float32),
        pltpu.SemaphoreType.DMA,
        pltpu.SemaphoreType.DMA,
    ],
)
def _score_kernel(acc4, u_h, i_h, out,
                  uid, iid, gxu, gxi, ru, ri, pb, sb, semU, semI):
    c = lax.axis_index("c")
    s = lax.axis_index("s")
    wid = s * 2 + c
    base = wid * 128
    pltpu.sync_copy(u_h.at[pl.ds(base, 128)], uid)
    pltpu.sync_copy(i_h.at[pl.ds(base, 128)], iid)
    _fill_zero_rows(pb, 128)

    for g in range(_NG):
        for k in range(8):
            sl = pl.ds(k * 16, 16)
            gxu[sl] = uid[sl] + g * _N
            gxi[sl] = iid[sl] + (g * _N + _NU)
        dU = pltpu.async_copy(acc4.at[gxu], ru, semU)
        dI = pltpu.async_copy(acc4.at[gxi], ri, semI)
        dU.wait()
        dI.wait()

        def row(r, _):
            pb[r, :] = pb[r, :] + ru[r, :] * ri[r, :]
            return 0
        lax.fori_loop(0, 128, row, 0)

    lanes = lax.iota(jnp.int32, 16)

    def sgrp(q, _):
        vec = jnp.zeros((16,), jnp.float32)
        for tt in range(16):
            r = q * 16 + tt
            sc = jnp.sum(pb[r, :]) * 0.0625
            vec = jnp.where(lanes == tt, sc, vec)
        sb[pl.ds(q * 16, 16)] = vec
        return 0
    lax.fori_loop(0, 8, sgrp, 0)
    pltpu.sync_copy(sb, out.at[pl.ds(base, 128)])


def kernel(U, V, u, i, adj_user, adj_item):
    e0g = (jnp.concatenate([U, V], axis=0)
           .reshape(_N, _NG, _GC).transpose(1, 0, 2).reshape(_NG * _N, _GC))
    npad = _TC * _CK - _NNZ
    au_p = jnp.concatenate([adj_user.astype(jnp.int32),
                            jnp.full((npad,), 101000, jnp.int32)])
    ai_p = jnp.concatenate([adj_item.astype(jnp.int32),
                            jnp.full((npad,), 51000, jnp.int32)])
    adj_c = jnp.stack([au_p.reshape(_TC, _CK), ai_p.reshape(_TC, _CK)], axis=1)
    dinv, dinv2, t = _prep_kernel(adj_c, e0g)
    acc = e0g
    for _ in range(_NLAYERS):
        t, acc = _layer_kernel(adj_c, t, acc, dinv, dinv2)
    return _score_kernel(acc, u.astype(jnp.int32), i.astype(jnp.int32))
